# Initial kernel scaffold; baseline (speedup 1.0000x reference)
import jax
import jax.numpy as jnp
from jax.experimental import pallas as pl


def kernel(x, edge_index, edge_type, batch, basis0, comp0, root0, bias0, basis1, comp1, root1, bias1, basis2, comp2, root2, bias2, basis3, comp3, root3, bias3, lin1_w, lin1_b, lin2_w, lin2_b):
    raise NotImplementedError("write your pallas kernel here")


# speedup vs baseline: 17.2176x; 17.2176x over previous
import functools

import jax
import jax.numpy as jnp
from jax import lax
from jax.experimental import pallas as pl
from jax.experimental.pallas import tpu as pltpu
from jax.experimental.pallas import tpu_sc as plsc

NC = 2
NS = 16
NW = NC * NS
CHUNK = 128

_mesh = plsc.VectorSubcoreMesh(core_axis_name="c", subcore_axis_name="s")


GRP = 16


def _fill_vmem(buf, nrows, val):
    v = jnp.full((16,), val, jnp.float32)

    def body(i, _):
        buf[i, pl.ds(0, 16)] = v
        buf[i, pl.ds(16, 16)] = v
        return 0

    lax.fori_loop(0, nrows, body, 0)


def _stripe_copy(src_block, acc_sh, out_h, s, c, stripe, drain):
    def body(k, _):
        rs = pl.multiple_of(s * stripe + k * CHUNK, CHUNK)
        if drain:
            pltpu.sync_copy(acc_sh.at[pl.ds(rs, CHUNK)], out_h.at[c, pl.ds(rs, CHUNK)])
        else:
            pltpu.sync_copy(src_block, acc_sh.at[pl.ds(rs, CHUNK)])
        return 0

    lax.fori_loop(0, stripe // CHUNK, body, 0)


def _make_edge_pass(nrp, ngrp):
    stripe = nrp // NS

    @functools.partial(
        pl.kernel,
        out_type=jax.ShapeDtypeStruct((NC, nrp, 32), jnp.float32),
        mesh=_mesh,
        scratch_types=[
            pltpu.VMEM((GRP, CHUNK), jnp.int32),
            pltpu.VMEM((GRP, CHUNK), jnp.int32),
            pltpu.VMEM((CHUNK, 32), jnp.float32),
            pltpu.VMEM_SHARED((nrp, 32), jnp.float32),
            pltpu.SemaphoreType.DMA,
        ],
        compiler_params=pltpu.CompilerParams(use_tc_tiling_on_sc=False),
    )
    def edge_pass(table_h, gidx_h, skey_h, out_h,
                  gidx_v, skey_v, rows_v, acc_sh, sem):
        c = lax.axis_index("c")
        s = lax.axis_index("s")
        wid = c * NS + s

        _fill_vmem(rows_v, CHUNK, 0.0)
        _stripe_copy(rows_v, acc_sh, out_h, s, c, stripe, drain=False)
        plsc.subcore_barrier()

        def group(g, _):
            pltpu.sync_copy(gidx_h.at[wid, g], gidx_v)
            pltpu.sync_copy(skey_h.at[wid, g], skey_v)

            def body(j, _):
                pltpu.async_copy(table_h.at[gidx_v.at[j]], rows_v, sem).wait()
                pltpu.sync_copy(rows_v, acc_sh.at[skey_v.at[j]], add=True)
                return 0

            lax.fori_loop(0, GRP, body, 0)
            return 0

        lax.fori_loop(0, ngrp, group, 0)

        plsc.subcore_barrier()
        _stripe_copy(rows_v, acc_sh, out_h, s, c, stripe, drain=True)

    return edge_pass


def _make_counts(nrp, ngrp):
    stripe = nrp // NS

    @functools.partial(
        pl.kernel,
        out_type=jax.ShapeDtypeStruct((NC, nrp, 32), jnp.float32),
        mesh=_mesh,
        scratch_types=[
            pltpu.VMEM((GRP, CHUNK), jnp.int32),
            pltpu.VMEM((CHUNK, 32), jnp.float32),
            pltpu.VMEM((CHUNK, 32), jnp.float32),
            pltpu.VMEM_SHARED((nrp, 32), jnp.float32),
        ],
        compiler_params=pltpu.CompilerParams(use_tc_tiling_on_sc=False),
    )
    def counts(skey_h, out_h, skey_v, ones_v, zero_v, acc_sh):
        c = lax.axis_index("c")
        s = lax.axis_index("s")
        wid = c * NS + s

        _fill_vmem(zero_v, CHUNK, 0.0)
        _fill_vmem(ones_v, CHUNK, 1.0)
        _stripe_copy(zero_v, acc_sh, out_h, s, c, stripe, drain=False)
        plsc.subcore_barrier()

        def group(g, _):
            pltpu.sync_copy(skey_h.at[wid, g], skey_v)

            def body(j, _):
                pltpu.sync_copy(ones_v, acc_sh.at[skey_v.at[j]], add=True)
                return 0

            lax.fori_loop(0, GRP, body, 0)
            return 0

        lax.fori_loop(0, ngrp, group, 0)

        plsc.subcore_barrier()
        _stripe_copy(zero_v, acc_sh, out_h, s, c, stripe, drain=True)

    return counts


def _xw_block(xb, comp_ref, xw_ref, r_dim, o):
    for r in range(r_dim):
        xw_ref[r] = comp_ref[r, 0] * xb[:, :o] + comp_ref[r, 1] * xb[:, o:]


def _xw0_kernel(h_ref, bf_ref, comp_ref, xw_ref, *, r_dim, o):
    xb = jnp.dot(h_ref[...], bf_ref[...], preferred_element_type=jnp.float32)
    _xw_block(xb, comp_ref, xw_ref, r_dim, o)


def _upd_kernel(p_ref, cnt_ref, h_ref, root_ref, bias_ref, bf_ref, comp_ref,
                hn_ref, xw_ref, *, r_dim, o, build_xw):
    agg = jnp.zeros(hn_ref.shape, jnp.float32)
    for r in range(r_dim):
        cnt = jnp.maximum(cnt_ref[0, r] + cnt_ref[1, r], 1.0)
        agg = agg + (p_ref[0, r] + p_ref[1, r]) / cnt
    hr = jnp.dot(h_ref[...], root_ref[...], preferred_element_type=jnp.float32)
    hn = jnp.tanh(agg + hr + bias_ref[...])
    hn_ref[...] = hn
    if build_xw:
        xb = jnp.dot(hn, bf_ref[...], preferred_element_type=jnp.float32)
        _xw_block(xb, comp_ref, xw_ref, r_dim, o)


def _head_kernel(u_ref, v_ref, w1u_ref, w1v_ref, b1_ref, w2_ref, b2_ref, out_ref):
    u = jnp.max(u_ref[...], axis=0)
    v = jnp.max(v_ref[...], axis=0)
    z = jnp.dot(u, w1u_ref[...], preferred_element_type=jnp.float32)
    z = z + jnp.dot(v, w1v_ref[...], preferred_element_type=jnp.float32)
    z = jnp.maximum(z + b1_ref[...], 0.0)
    z = jnp.dot(z, w2_ref[...], preferred_element_type=jnp.float32) + b2_ref[...]
    m = jnp.max(z, axis=-1, keepdims=True)
    lse = jnp.log(jnp.sum(jnp.exp(z - m), axis=-1, keepdims=True)) + m
    out_ref[...] = z - lse


def _build_xw0(x, bf, comp, n, r_dim, o, nb_rows):
    grid = n // nb_rows
    return pl.pallas_call(
        functools.partial(_xw0_kernel, r_dim=r_dim, o=o),
        grid=(grid,),
        in_specs=[
            pl.BlockSpec((nb_rows, x.shape[1]), lambda i: (i, 0)),
            pl.BlockSpec((bf.shape[0], bf.shape[1]), lambda i: (0, 0)),
            pl.BlockSpec(memory_space=pltpu.SMEM),
        ],
        out_specs=pl.BlockSpec((r_dim, nb_rows, o), lambda i: (0, i, 0)),
        out_shape=jax.ShapeDtypeStruct((r_dim, n, o), jnp.float32),
    )(x, bf, comp)


def _layer_update(p, cnt, h, root, bias, bf, comp, n, r_dim, o, nb_rows, build_xw):
    grid = n // nb_rows
    din = h.shape[1]
    kern = functools.partial(_upd_kernel, r_dim=r_dim, o=o, build_xw=build_xw)
    out_shapes = [jax.ShapeDtypeStruct((n, o), jnp.float32),
                  jax.ShapeDtypeStruct((r_dim, n, o), jnp.float32)]
    out_specs = [pl.BlockSpec((nb_rows, o), lambda i: (i, 0)),
                 pl.BlockSpec((r_dim, nb_rows, o), lambda i: (0, i, 0))]
    return pl.pallas_call(
        kern,
        grid=(grid,),
        in_specs=[
            pl.BlockSpec((2, r_dim, nb_rows, o), lambda i: (0, 0, i, 0)),
            pl.BlockSpec((2, r_dim, nb_rows, o), lambda i: (0, 0, i, 0)),
            pl.BlockSpec((nb_rows, din), lambda i: (i, 0)),
            pl.BlockSpec((din, o), lambda i: (0, 0)),
            pl.BlockSpec((1, o), lambda i: (0, 0)),
            pl.BlockSpec((o, bf.shape[1]), lambda i: (0, 0)),
            pl.BlockSpec(memory_space=pltpu.SMEM),
        ],
        out_specs=out_specs,
        out_shape=out_shapes,
    )(p, cnt, h, root, bias, bf, comp)


def _head(u, v, w1u, w1v, b1, w2, b2, b_rows):
    return pl.pallas_call(
        _head_kernel,
        out_shape=jax.ShapeDtypeStruct((b_rows, w2.shape[1]), jnp.float32),
    )(u, v, w1u, w1v, b1, w2, b2)


def kernel(x, edge_index, edge_type, batch, basis0, comp0, root0, bias0,
           basis1, comp1, root1, bias1, basis2, comp2, root2, bias2,
           basis3, comp3, root3, bias3, lin1_w, lin1_b, lin2_w, lin2_b):
    n, din = x.shape
    e = edge_type.shape[0]
    r_dim, nb = comp0.shape
    o = basis0.shape[2]
    b_rows = 100

    nr = n * r_dim
    nrp = ((nr + 1 + NS * CHUNK - 1) // (NS * CHUNK)) * (NS * CHUNK)
    trash = nr

    gblk = GRP * CHUNK
    epw = ((e + NW * gblk - 1) // (NW * gblk)) * gblk
    ngrp = epw // gblk
    e_pad = epw * NW

    src, dst = edge_index[0], edge_index[1]
    gidx = edge_type * n + src
    skey = edge_type * n + dst
    pad = e_pad - e
    gidx4 = jnp.concatenate([gidx, jnp.zeros((pad,), jnp.int32)]).reshape(NW, ngrp, GRP, CHUNK)
    skey4 = jnp.concatenate([skey, jnp.full((pad,), trash, jnp.int32)]).reshape(NW, ngrp, GRP, CHUNK)

    edge_pass = _make_edge_pass(nrp, ngrp)
    counts_fn = _make_counts(nrp, ngrp)

    cnt2 = counts_fn(skey4)
    cnt = cnt2[:, :nr].reshape(2, r_dim, n, o)

    def bf(basis):
        return jnp.transpose(basis, (1, 0, 2)).reshape(basis.shape[1], nb * o)

    nb_rows = 2000
    params = [(basis0, comp0, root0, bias0), (basis1, comp1, root1, bias1),
              (basis2, comp2, root2, bias2), (basis3, comp3, root3, bias3)]

    xw = _build_xw0(x, bf(basis0), comp0, n, r_dim, o, nb_rows)
    h = x
    hs = []
    for l in range(4):
        p2 = edge_pass(xw.reshape(nr, o), gidx4, skey4)
        p = p2[:, :nr].reshape(2, r_dim, n, o)
        _, _, root, bias = params[l]
        build = l < 3
        nxt_basis, nxt_comp = (params[l + 1][0], params[l + 1][1]) if build else (params[l][0], params[l][1])
        hn, xw_next = _layer_update(p, cnt, h, root, bias.reshape(1, o),
                                    bf(nxt_basis), nxt_comp, n, r_dim, o,
                                    nb_rows, build)
        hs.append(hn)
        h = hn
        xw = xw_next

    users_idx = jnp.nonzero(x[:, 0] == 1, size=b_rows)[0]
    items_idx = jnp.nonzero(x[:, 1] == 1, size=b_rows)[0]
    hstack = jnp.stack(hs, axis=0)
    u = jnp.take(hstack, users_idx, axis=1)
    v = jnp.take(hstack, items_idx, axis=1)
    w1t = lin1_w.T
    half = w1t.shape[0] // 2
    return _head(u, v, w1t[:half], w1t[half:], lin1_b.reshape(1, -1),
                 lin2_w.T, lin2_b.reshape(1, -1), b_rows)


# speedup vs baseline: 19.4776x; 1.1313x over previous
import functools

import jax
import jax.numpy as jnp
from jax import lax
from jax.experimental import pallas as pl
from jax.experimental.pallas import tpu as pltpu
from jax.experimental.pallas import tpu_sc as plsc

NC = 2
NS = 16
NW = NC * NS
CHUNK = 128

_mesh = plsc.VectorSubcoreMesh(core_axis_name="c", subcore_axis_name="s")


GRP = 16


def _fill_vmem(buf, nrows, val):
    v = jnp.full((16,), val, jnp.float32)

    def body(i, _):
        buf[i, pl.ds(0, 16)] = v
        buf[i, pl.ds(16, 16)] = v
        return 0

    lax.fori_loop(0, nrows, body, 0)


def _stripe_copy(src_block, acc_sh, out_h, s, c, stripe, drain):
    def body(k, _):
        rs = pl.multiple_of(s * stripe + k * CHUNK, CHUNK)
        if drain:
            pltpu.sync_copy(acc_sh.at[pl.ds(rs, CHUNK)], out_h.at[c, pl.ds(rs, CHUNK)])
        else:
            pltpu.sync_copy(src_block, acc_sh.at[pl.ds(rs, CHUNK)])
        return 0

    lax.fori_loop(0, stripe // CHUNK, body, 0)


NBUF = 4


def _make_edge_pass(nrp, ngrp):
    stripe = nrp // NS

    @functools.partial(
        pl.kernel,
        out_type=jax.ShapeDtypeStruct((NC, nrp, 32), jnp.float32),
        mesh=_mesh,
        scratch_types=[
            pltpu.VMEM((GRP, CHUNK), jnp.int32),
            pltpu.VMEM((GRP, CHUNK), jnp.int32),
            pltpu.VMEM((NBUF, CHUNK, 32), jnp.float32),
            pltpu.VMEM_SHARED((nrp, 32), jnp.float32),
        ] + [pltpu.SemaphoreType.DMA] * NBUF,
        compiler_params=pltpu.CompilerParams(use_tc_tiling_on_sc=False),
    )
    def edge_pass(table_h, gidx_h, skey_h, out_h,
                  gidx_v, skey_v, rows_v, acc_sh, *sems):
        c = lax.axis_index("c")
        s = lax.axis_index("s")
        wid = c * NS + s

        _fill_vmem(rows_v.at[0], CHUNK, 0.0)
        _stripe_copy(rows_v.at[0], acc_sh, out_h, s, c, stripe, drain=False)
        plsc.subcore_barrier()

        def group(g, _):
            pltpu.sync_copy(gidx_h.at[wid, g], gidx_v)
            pltpu.sync_copy(skey_h.at[wid, g], skey_v)

            for b in range(NBUF):
                pltpu.async_copy(table_h.at[gidx_v.at[b]], rows_v.at[b], sems[b])

            def pipe(jj, _):
                for b in range(NBUF):
                    ch = jj * NBUF + b
                    pltpu.make_async_copy(
                        table_h.at[pl.ds(0, CHUNK)], rows_v.at[b], sems[b]
                    ).wait()
                    pltpu.sync_copy(rows_v.at[b], acc_sh.at[skey_v.at[ch]],
                                    add=True)
                    pltpu.async_copy(table_h.at[gidx_v.at[ch + NBUF]],
                                     rows_v.at[b], sems[b])
                return 0

            lax.fori_loop(0, GRP // NBUF - 1, pipe, 0)

            for b in range(NBUF):
                ch = GRP - NBUF + b
                pltpu.make_async_copy(
                    table_h.at[pl.ds(0, CHUNK)], rows_v.at[b], sems[b]
                ).wait()
                pltpu.sync_copy(rows_v.at[b], acc_sh.at[skey_v.at[ch]],
                                add=True)
            return 0

        lax.fori_loop(0, ngrp, group, 0)

        plsc.subcore_barrier()
        _stripe_copy(rows_v.at[0], acc_sh, out_h, s, c, stripe, drain=True)

    return edge_pass


def _make_counts(nrp, ngrp):
    stripe = nrp // NS

    @functools.partial(
        pl.kernel,
        out_type=jax.ShapeDtypeStruct((NC, nrp, 32), jnp.float32),
        mesh=_mesh,
        scratch_types=[
            pltpu.VMEM((GRP, CHUNK), jnp.int32),
            pltpu.VMEM((CHUNK, 32), jnp.float32),
            pltpu.VMEM((CHUNK, 32), jnp.float32),
            pltpu.VMEM_SHARED((nrp, 32), jnp.float32),
        ],
        compiler_params=pltpu.CompilerParams(use_tc_tiling_on_sc=False),
    )
    def counts(skey_h, out_h, skey_v, ones_v, zero_v, acc_sh):
        c = lax.axis_index("c")
        s = lax.axis_index("s")
        wid = c * NS + s

        _fill_vmem(zero_v, CHUNK, 0.0)
        _fill_vmem(ones_v, CHUNK, 1.0)
        _stripe_copy(zero_v, acc_sh, out_h, s, c, stripe, drain=False)
        plsc.subcore_barrier()

        def group(g, _):
            pltpu.sync_copy(skey_h.at[wid, g], skey_v)

            def body(j, _):
                pltpu.sync_copy(ones_v, acc_sh.at[skey_v.at[j]], add=True)
                return 0

            lax.fori_loop(0, GRP, body, 0)
            return 0

        lax.fori_loop(0, ngrp, group, 0)

        plsc.subcore_barrier()
        _stripe_copy(zero_v, acc_sh, out_h, s, c, stripe, drain=True)

    return counts


def _xw_block(xb, comp_ref, xw_ref, r_dim, o):
    for r in range(r_dim):
        xw_ref[r] = comp_ref[r, 0] * xb[:, :o] + comp_ref[r, 1] * xb[:, o:]


def _xw0_kernel(h_ref, bf_ref, comp_ref, xw_ref, *, r_dim, o):
    xb = jnp.dot(h_ref[...], bf_ref[...], preferred_element_type=jnp.float32)
    _xw_block(xb, comp_ref, xw_ref, r_dim, o)


def _upd_kernel(p_ref, cnt_ref, h_ref, root_ref, bias_ref, bf_ref, comp_ref,
                hn_ref, xw_ref, *, r_dim, o, build_xw):
    agg = jnp.zeros(hn_ref.shape, jnp.float32)
    for r in range(r_dim):
        cnt = jnp.maximum(cnt_ref[0, r] + cnt_ref[1, r], 1.0)
        agg = agg + (p_ref[0, r] + p_ref[1, r]) / cnt
    hr = jnp.dot(h_ref[...], root_ref[...], preferred_element_type=jnp.float32)
    hn = jnp.tanh(agg + hr + bias_ref[...])
    hn_ref[...] = hn
    if build_xw:
        xb = jnp.dot(hn, bf_ref[...], preferred_element_type=jnp.float32)
        _xw_block(xb, comp_ref, xw_ref, r_dim, o)


def _head_kernel(u_ref, v_ref, w1u_ref, w1v_ref, b1_ref, w2_ref, b2_ref, out_ref):
    u = jnp.max(u_ref[...], axis=0)
    v = jnp.max(v_ref[...], axis=0)
    z = jnp.dot(u, w1u_ref[...], preferred_element_type=jnp.float32)
    z = z + jnp.dot(v, w1v_ref[...], preferred_element_type=jnp.float32)
    z = jnp.maximum(z + b1_ref[...], 0.0)
    z = jnp.dot(z, w2_ref[...], preferred_element_type=jnp.float32) + b2_ref[...]
    m = jnp.max(z, axis=-1, keepdims=True)
    lse = jnp.log(jnp.sum(jnp.exp(z - m), axis=-1, keepdims=True)) + m
    out_ref[...] = z - lse


def _build_xw0(x, bf, comp, n, r_dim, o, nb_rows):
    grid = n // nb_rows
    return pl.pallas_call(
        functools.partial(_xw0_kernel, r_dim=r_dim, o=o),
        grid=(grid,),
        in_specs=[
            pl.BlockSpec((nb_rows, x.shape[1]), lambda i: (i, 0)),
            pl.BlockSpec((bf.shape[0], bf.shape[1]), lambda i: (0, 0)),
            pl.BlockSpec(memory_space=pltpu.SMEM),
        ],
        out_specs=pl.BlockSpec((r_dim, nb_rows, o), lambda i: (0, i, 0)),
        out_shape=jax.ShapeDtypeStruct((r_dim, n, o), jnp.float32),
    )(x, bf, comp)


def _layer_update(p, cnt, h, root, bias, bf, comp, n, r_dim, o, nb_rows, build_xw):
    grid = n // nb_rows
    din = h.shape[1]
    kern = functools.partial(_upd_kernel, r_dim=r_dim, o=o, build_xw=build_xw)
    out_shapes = [jax.ShapeDtypeStruct((n, o), jnp.float32),
                  jax.ShapeDtypeStruct((r_dim, n, o), jnp.float32)]
    out_specs = [pl.BlockSpec((nb_rows, o), lambda i: (i, 0)),
                 pl.BlockSpec((r_dim, nb_rows, o), lambda i: (0, i, 0))]
    return pl.pallas_call(
        kern,
        grid=(grid,),
        in_specs=[
            pl.BlockSpec((2, r_dim, nb_rows, o), lambda i: (0, 0, i, 0)),
            pl.BlockSpec((2, r_dim, nb_rows, o), lambda i: (0, 0, i, 0)),
            pl.BlockSpec((nb_rows, din), lambda i: (i, 0)),
            pl.BlockSpec((din, o), lambda i: (0, 0)),
            pl.BlockSpec((1, o), lambda i: (0, 0)),
            pl.BlockSpec((o, bf.shape[1]), lambda i: (0, 0)),
            pl.BlockSpec(memory_space=pltpu.SMEM),
        ],
        out_specs=out_specs,
        out_shape=out_shapes,
    )(p, cnt, h, root, bias, bf, comp)


def _head(u, v, w1u, w1v, b1, w2, b2, b_rows):
    return pl.pallas_call(
        _head_kernel,
        out_shape=jax.ShapeDtypeStruct((b_rows, w2.shape[1]), jnp.float32),
    )(u, v, w1u, w1v, b1, w2, b2)


def kernel(x, edge_index, edge_type, batch, basis0, comp0, root0, bias0,
           basis1, comp1, root1, bias1, basis2, comp2, root2, bias2,
           basis3, comp3, root3, bias3, lin1_w, lin1_b, lin2_w, lin2_b):
    n, din = x.shape
    e = edge_type.shape[0]
    r_dim, nb = comp0.shape
    o = basis0.shape[2]
    b_rows = 100

    nr = n * r_dim
    nrp = ((nr + 1 + NS * CHUNK - 1) // (NS * CHUNK)) * (NS * CHUNK)
    trash = nr

    gblk = GRP * CHUNK
    epw = ((e + NW * gblk - 1) // (NW * gblk)) * gblk
    ngrp = epw // gblk
    e_pad = epw * NW

    src, dst = edge_index[0], edge_index[1]
    gidx = edge_type * n + src
    skey = edge_type * n + dst
    pad = e_pad - e
    gidx4 = jnp.concatenate([gidx, jnp.zeros((pad,), jnp.int32)]).reshape(NW, ngrp, GRP, CHUNK)
    skey4 = jnp.concatenate([skey, jnp.full((pad,), trash, jnp.int32)]).reshape(NW, ngrp, GRP, CHUNK)

    edge_pass = _make_edge_pass(nrp, ngrp)
    counts_fn = _make_counts(nrp, ngrp)

    cnt2 = counts_fn(skey4)
    cnt = cnt2[:, :nr].reshape(2, r_dim, n, o)

    def bf(basis):
        return jnp.transpose(basis, (1, 0, 2)).reshape(basis.shape[1], nb * o)

    nb_rows = 2000
    params = [(basis0, comp0, root0, bias0), (basis1, comp1, root1, bias1),
              (basis2, comp2, root2, bias2), (basis3, comp3, root3, bias3)]

    xw = _build_xw0(x, bf(basis0), comp0, n, r_dim, o, nb_rows)
    h = x
    hs = []
    for l in range(4):
        p2 = edge_pass(xw.reshape(nr, o), gidx4, skey4)
        p = p2[:, :nr].reshape(2, r_dim, n, o)
        _, _, root, bias = params[l]
        build = l < 3
        nxt_basis, nxt_comp = (params[l + 1][0], params[l + 1][1]) if build else (params[l][0], params[l][1])
        hn, xw_next = _layer_update(p, cnt, h, root, bias.reshape(1, o),
                                    bf(nxt_basis), nxt_comp, n, r_dim, o,
                                    nb_rows, build)
        hs.append(hn)
        h = hn
        xw = xw_next

    users_idx = jnp.nonzero(x[:, 0] == 1, size=b_rows)[0]
    items_idx = jnp.nonzero(x[:, 1] == 1, size=b_rows)[0]
    hstack = jnp.stack(hs, axis=0)
    u = jnp.take(hstack, users_idx, axis=1)
    v = jnp.take(hstack, items_idx, axis=1)
    w1t = lin1_w.T
    half = w1t.shape[0] // 2
    return _head(u, v, w1t[:half], w1t[half:], lin1_b.reshape(1, -1),
                 lin2_w.T, lin2_b.reshape(1, -1), b_rows)


# speedup vs baseline: 20.9701x; 1.0766x over previous
import functools

import jax
import jax.numpy as jnp
from jax import lax
from jax.experimental import pallas as pl
from jax.experimental.pallas import tpu as pltpu
from jax.experimental.pallas import tpu_sc as plsc

NC = 2
NS = 16
NW = NC * NS
CHUNK = 128

_mesh = plsc.VectorSubcoreMesh(core_axis_name="c", subcore_axis_name="s")


GRP = 16


def _fill_vmem(buf, nrows, val):
    v = jnp.full((16,), val, jnp.float32)

    def body(i, _):
        buf[i, pl.ds(0, 16)] = v
        buf[i, pl.ds(16, 16)] = v
        return 0

    lax.fori_loop(0, nrows, body, 0)


def _stripe_copy(src_block, acc_sh, out_h, s, c, stripe, drain):
    def body(k, _):
        rs = pl.multiple_of(s * stripe + k * CHUNK, CHUNK)
        if drain:
            pltpu.sync_copy(acc_sh.at[pl.ds(rs, CHUNK)], out_h.at[c, pl.ds(rs, CHUNK)])
        else:
            pltpu.sync_copy(src_block, acc_sh.at[pl.ds(rs, CHUNK)])
        return 0

    lax.fori_loop(0, stripe // CHUNK, body, 0)


NBUF = 4


def _make_edge_pass(nrp, ngrp):
    stripe = nrp // NS

    @functools.partial(
        pl.kernel,
        out_type=jax.ShapeDtypeStruct((NC, nrp, 32), jnp.float32),
        mesh=_mesh,
        scratch_types=[
            pltpu.VMEM((GRP, CHUNK), jnp.int32),
            pltpu.VMEM((GRP, CHUNK), jnp.int32),
            pltpu.VMEM((NBUF, CHUNK, 32), jnp.float32),
            pltpu.VMEM_SHARED((nrp, 32), jnp.float32),
        ] + [pltpu.SemaphoreType.DMA] * NBUF,
        compiler_params=pltpu.CompilerParams(use_tc_tiling_on_sc=False),
    )
    def edge_pass(table_h, gidx_h, skey_h, out_h,
                  gidx_v, skey_v, rows_v, acc_sh, *sems):
        c = lax.axis_index("c")
        s = lax.axis_index("s")
        wid = c * NS + s

        _fill_vmem(rows_v.at[0], CHUNK, 0.0)
        _stripe_copy(rows_v.at[0], acc_sh, out_h, s, c, stripe, drain=False)
        plsc.subcore_barrier()

        def group(g, _):
            pltpu.sync_copy(gidx_h.at[wid, g], gidx_v)
            pltpu.sync_copy(skey_h.at[wid, g], skey_v)

            for b in range(NBUF):
                pltpu.async_copy(table_h.at[gidx_v.at[b]], rows_v.at[b], sems[b])

            def pipe(jj, _):
                for b in range(NBUF):
                    ch = jj * NBUF + b
                    pltpu.make_async_copy(
                        table_h.at[pl.ds(0, CHUNK)], rows_v.at[b], sems[b]
                    ).wait()
                    pltpu.sync_copy(rows_v.at[b], acc_sh.at[skey_v.at[ch]],
                                    add=True)
                    pltpu.async_copy(table_h.at[gidx_v.at[ch + NBUF]],
                                     rows_v.at[b], sems[b])
                return 0

            lax.fori_loop(0, GRP // NBUF - 1, pipe, 0)

            for b in range(NBUF):
                ch = GRP - NBUF + b
                pltpu.make_async_copy(
                    table_h.at[pl.ds(0, CHUNK)], rows_v.at[b], sems[b]
                ).wait()
                pltpu.sync_copy(rows_v.at[b], acc_sh.at[skey_v.at[ch]],
                                add=True)
            return 0

        lax.fori_loop(0, ngrp, group, 0)

        plsc.subcore_barrier()
        _stripe_copy(rows_v.at[0], acc_sh, out_h, s, c, stripe, drain=True)

    return edge_pass


def _make_counts(nrp, ngrp):
    stripe = nrp // NS

    @functools.partial(
        pl.kernel,
        out_type=jax.ShapeDtypeStruct((NC, nrp, 32), jnp.float32),
        mesh=_mesh,
        scratch_types=[
            pltpu.VMEM((GRP, CHUNK), jnp.int32),
            pltpu.VMEM((CHUNK, 32), jnp.float32),
            pltpu.VMEM((CHUNK, 32), jnp.float32),
            pltpu.VMEM_SHARED((nrp, 32), jnp.float32),
        ],
        compiler_params=pltpu.CompilerParams(use_tc_tiling_on_sc=False),
    )
    def counts(skey_h, out_h, skey_v, ones_v, zero_v, acc_sh):
        c = lax.axis_index("c")
        s = lax.axis_index("s")
        wid = c * NS + s

        _fill_vmem(zero_v, CHUNK, 0.0)
        _fill_vmem(ones_v, CHUNK, 1.0)
        _stripe_copy(zero_v, acc_sh, out_h, s, c, stripe, drain=False)
        plsc.subcore_barrier()

        def group(g, _):
            pltpu.sync_copy(skey_h.at[wid, g], skey_v)

            def body(j, _):
                pltpu.sync_copy(ones_v, acc_sh.at[skey_v.at[j]], add=True)
                return 0

            lax.fori_loop(0, GRP, body, 0)
            return 0

        lax.fori_loop(0, ngrp, group, 0)

        plsc.subcore_barrier()
        _stripe_copy(zero_v, acc_sh, out_h, s, c, stripe, drain=True)

    return counts


def _xw_block(xb, comp_ref, xw_ref, r_dim, o):
    for r in range(r_dim):
        xw_ref[r] = comp_ref[r, 0] * xb[:, :o] + comp_ref[r, 1] * xb[:, o:]


def _xw0_kernel(h_ref, bf_ref, comp_ref, xw_ref, *, r_dim, o):
    xb = jnp.dot(h_ref[...], bf_ref[...], preferred_element_type=jnp.float32)
    _xw_block(xb, comp_ref, xw_ref, r_dim, o)


def _upd_kernel(p_ref, cnt_ref, h_ref, root_ref, bias_ref, bf_ref, comp_ref,
                hn_ref, xw_ref, *, r_dim, o, build_xw):
    agg = jnp.zeros(hn_ref.shape, jnp.float32)
    for r in range(r_dim):
        cnt = jnp.maximum(cnt_ref[0, r] + cnt_ref[1, r], 1.0)
        agg = agg + (p_ref[0, r] + p_ref[1, r]) / cnt
    hr = jnp.dot(h_ref[...], root_ref[...], preferred_element_type=jnp.float32)
    hn = jnp.tanh(agg + hr + bias_ref[...])
    hn_ref[...] = hn
    if build_xw:
        xb = jnp.dot(hn, bf_ref[...], preferred_element_type=jnp.float32)
        _xw_block(xb, comp_ref, xw_ref, r_dim, o)


def _head_kernel(u_ref, v_ref, w1u_ref, w1v_ref, b1_ref, w2_ref, b2_ref, out_ref):
    u = jnp.max(u_ref[...], axis=0)
    v = jnp.max(v_ref[...], axis=0)
    z = jnp.dot(u, w1u_ref[...], preferred_element_type=jnp.float32)
    z = z + jnp.dot(v, w1v_ref[...], preferred_element_type=jnp.float32)
    z = jnp.maximum(z + b1_ref[...], 0.0)
    z = jnp.dot(z, w2_ref[...], preferred_element_type=jnp.float32) + b2_ref[...]
    m = jnp.max(z, axis=-1, keepdims=True)
    lse = jnp.log(jnp.sum(jnp.exp(z - m), axis=-1, keepdims=True)) + m
    out_ref[...] = z - lse


def _build_xw0(x, bf, comp, n, r_dim, o, nb_rows):
    grid = n // nb_rows
    return pl.pallas_call(
        functools.partial(_xw0_kernel, r_dim=r_dim, o=o),
        grid=(grid,),
        in_specs=[
            pl.BlockSpec((nb_rows, x.shape[1]), lambda i: (i, 0)),
            pl.BlockSpec((bf.shape[0], bf.shape[1]), lambda i: (0, 0)),
            pl.BlockSpec(memory_space=pltpu.SMEM),
        ],
        out_specs=pl.BlockSpec((r_dim, nb_rows, o), lambda i: (0, i, 0)),
        out_shape=jax.ShapeDtypeStruct((r_dim, n, o), jnp.float32),
    )(x, bf, comp)


def _layer_update(p, cnt, h, root, bias, bf, comp, n, r_dim, o, nb_rows, build_xw):
    grid = n // nb_rows
    din = h.shape[1]
    kern = functools.partial(_upd_kernel, r_dim=r_dim, o=o, build_xw=build_xw)
    out_shapes = [jax.ShapeDtypeStruct((n, o), jnp.float32),
                  jax.ShapeDtypeStruct((r_dim, n, o), jnp.float32)]
    out_specs = [pl.BlockSpec((nb_rows, o), lambda i: (i, 0)),
                 pl.BlockSpec((r_dim, nb_rows, o), lambda i: (0, i, 0))]
    return pl.pallas_call(
        kern,
        grid=(grid,),
        in_specs=[
            pl.BlockSpec((2, r_dim, nb_rows, o), lambda i: (0, 0, i, 0)),
            pl.BlockSpec((2, r_dim, nb_rows, o), lambda i: (0, 0, i, 0)),
            pl.BlockSpec((nb_rows, din), lambda i: (i, 0)),
            pl.BlockSpec((din, o), lambda i: (0, 0)),
            pl.BlockSpec((1, o), lambda i: (0, 0)),
            pl.BlockSpec((o, bf.shape[1]), lambda i: (0, 0)),
            pl.BlockSpec(memory_space=pltpu.SMEM),
        ],
        out_specs=out_specs,
        out_shape=out_shapes,
    )(p, cnt, h, root, bias, bf, comp)


def _head(u, v, w1u, w1v, b1, w2, b2, b_rows):
    return pl.pallas_call(
        _head_kernel,
        out_shape=jax.ShapeDtypeStruct((b_rows, w2.shape[1]), jnp.float32),
    )(u, v, w1u, w1v, b1, w2, b2)


def kernel(x, edge_index, edge_type, batch, basis0, comp0, root0, bias0,
           basis1, comp1, root1, bias1, basis2, comp2, root2, bias2,
           basis3, comp3, root3, bias3, lin1_w, lin1_b, lin2_w, lin2_b):
    n, din = x.shape
    e = edge_type.shape[0]
    r_dim, nb = comp0.shape
    o = basis0.shape[2]
    b_rows = 100

    nr = n * r_dim
    nrp = ((nr + 1 + NS * CHUNK - 1) // (NS * CHUNK)) * (NS * CHUNK)
    trash = nr

    gblk = GRP * CHUNK
    epw = ((e + NW * gblk - 1) // (NW * gblk)) * gblk
    ngrp = epw // gblk
    e_pad = epw * NW

    src, dst = edge_index[0], edge_index[1]
    gidx = edge_type * n + src
    skey = edge_type * n + dst
    pad = e_pad - e
    def interleave(a):
        return a.reshape(-1, NW).T.reshape(NW, ngrp, GRP, CHUNK)

    gidx4 = interleave(jnp.concatenate([gidx, jnp.zeros((pad,), jnp.int32)]))
    skey4 = interleave(jnp.concatenate([skey, jnp.full((pad,), trash, jnp.int32)]))

    edge_pass = _make_edge_pass(nrp, ngrp)
    counts_fn = _make_counts(nrp, ngrp)

    cnt2 = counts_fn(skey4)
    cnt = cnt2[:, :nr].reshape(2, r_dim, n, o)

    def bf(basis):
        return jnp.transpose(basis, (1, 0, 2)).reshape(basis.shape[1], nb * o)

    nb_rows = 2000
    params = [(basis0, comp0, root0, bias0), (basis1, comp1, root1, bias1),
              (basis2, comp2, root2, bias2), (basis3, comp3, root3, bias3)]

    xw = _build_xw0(x, bf(basis0), comp0, n, r_dim, o, nb_rows)
    h = x
    hs = []
    for l in range(4):
        p2 = edge_pass(xw.reshape(nr, o), gidx4, skey4)
        p = p2[:, :nr].reshape(2, r_dim, n, o)
        _, _, root, bias = params[l]
        build = l < 3
        nxt_basis, nxt_comp = (params[l + 1][0], params[l + 1][1]) if build else (params[l][0], params[l][1])
        hn, xw_next = _layer_update(p, cnt, h, root, bias.reshape(1, o),
                                    bf(nxt_basis), nxt_comp, n, r_dim, o,
                                    nb_rows, build)
        hs.append(hn)
        h = hn
        xw = xw_next

    users_idx = jnp.nonzero(x[:, 0] == 1, size=b_rows)[0]
    items_idx = jnp.nonzero(x[:, 1] == 1, size=b_rows)[0]
    hstack = jnp.stack(hs, axis=0)
    u = jnp.take(hstack, users_idx, axis=1)
    v = jnp.take(hstack, items_idx, axis=1)
    w1t = lin1_w.T
    half = w1t.shape[0] // 2
    return _head(u, v, w1t[:half], w1t[half:], lin1_b.reshape(1, -1),
                 lin2_w.T, lin2_b.reshape(1, -1), b_rows)
